# baseline (speedup 1.0000x reference)
import functools

import jax
import jax.numpy as jnp
from jax import lax
from jax.experimental import pallas as pl
from jax.experimental.pallas import tpu as pltpu
from jax.experimental.pallas import tpu_sc as plsc

NC = 2
NS = 16
NW = NC * NS
L = 16
HIDDEN = 128
EPS = 1e-12
CHUNK = 128
GROUPS = CHUNK // L


def _body(idx_hbm, table_hbm, gamma_hbm, beta_hbm, norm_out, raw_out,
          idx_v, raw_v, norm_v, gamma_v, beta_v, sem):
    chunks = idx_hbm.shape[1]
    wid = lax.axis_index("s") * NC + lax.axis_index("c")
    rows_per_worker = chunks * CHUNK
    base_w = wid * rows_per_worker

    pltpu.sync_copy(idx_hbm.at[wid], idx_v)
    pltpu.sync_copy(gamma_hbm, gamma_v)
    pltpu.sync_copy(beta_hbm, beta_v)

    iota = lax.iota(jnp.int32, L)
    row_ids = [g * L + iota for g in range(GROUPS)]
    inv_h = jnp.float32(1.0 / HIDDEN)

    def chunk_body(i, carry):
        pltpu.async_copy(table_hbm.at[idx_v.at[i]], raw_v, sem).wait()
        base = base_w + i * CHUNK
        pltpu.sync_copy(raw_v, raw_out.at[pl.ds(base, CHUNK)])

        def col_stats(c, sc):
            sums, sumsqs = sc
            colv = jnp.full((L,), c, dtype=jnp.int32)
            ns, nq = [], []
            for g in range(GROUPS):
                x = plsc.load_gather(raw_v, [row_ids[g], colv])
                ns.append(sums[g] + x)
                nq.append(sumsqs[g] + x * x)
            return tuple(ns), tuple(nq)

        zeros = tuple(jnp.zeros((L,), jnp.float32) for _ in range(GROUPS))
        sums, sumsqs = lax.fori_loop(0, HIDDEN, col_stats, (zeros, zeros))

        means, scales = [], []
        for g in range(GROUPS):
            m = sums[g] * inv_h
            var = sumsqs[g] * inv_h - m * m
            a = var + jnp.float32(EPS)
            ib = plsc.bitcast(a, jnp.int32)
            magic = jnp.full((L,), 0x5F3759DF, dtype=jnp.int32)
            one = jnp.full((L,), 1, dtype=jnp.int32)
            y = plsc.bitcast(magic - lax.shift_right_logical(ib, one),
                             jnp.float32)
            half_a = a * jnp.float32(0.5)
            for _ in range(3):
                y = y * (jnp.float32(1.5) - half_a * y * y)
            means.append(m)
            scales.append(y)

        def col_norm(c, cc):
            colv = jnp.full((L,), c, dtype=jnp.int32)
            gc = plsc.load_gather(gamma_v, [colv])
            bc = plsc.load_gather(beta_v, [colv])
            for g in range(GROUPS):
                x = plsc.load_gather(raw_v, [row_ids[g], colv])
                y = (x - means[g]) * scales[g] * gc + bc
                plsc.store_scatter(norm_v, [row_ids[g], colv], y)
            return cc

        lax.fori_loop(0, HIDDEN, col_norm, 0)
        pltpu.sync_copy(norm_v, norm_out.at[pl.ds(base, CHUNK)])
        return carry

    lax.fori_loop(0, chunks, chunk_body, 0)


def kernel(eye_feature, table, ln_gamma, ln_beta):
    s0, s1 = eye_feature.shape
    b = s0 * s1
    assert b % (NW * CHUNK) == 0
    chunks = b // (NW * CHUNK)
    hidden = table.shape[1]

    idx = eye_feature.reshape(NW, chunks, CHUNK).astype(jnp.int32)

    run = pl.kernel(
        _body,
        out_type=(
            jax.ShapeDtypeStruct((b, hidden), jnp.float32),
            jax.ShapeDtypeStruct((b, hidden), jnp.float32),
        ),
        mesh=plsc.VectorSubcoreMesh(core_axis_name="c", subcore_axis_name="s"),
        compiler_params=pltpu.CompilerParams(needs_layout_passes=False),
        scratch_types=[
            pltpu.VMEM((chunks, CHUNK), jnp.int32),
            pltpu.VMEM((CHUNK, hidden), jnp.float32),
            pltpu.VMEM((CHUNK, hidden), jnp.float32),
            pltpu.VMEM((hidden,), jnp.float32),
            pltpu.VMEM((hidden,), jnp.float32),
            pltpu.SemaphoreType.DMA,
        ],
    )
    norm, raw = run(idx, table, ln_gamma, ln_beta)
    return (norm.reshape(s0, s1, hidden), raw.reshape(s0, s1, hidden))


# speedup vs baseline: 1.0713x; 1.0713x over previous
import jax
import jax.numpy as jnp
from jax import lax
from jax.experimental import pallas as pl
from jax.experimental.pallas import tpu as pltpu
from jax.experimental.pallas import tpu_sc as plsc

NC = 2
NS = 16
NW = NC * NS
L = 16
HIDDEN = 128
EPS = 1e-12
CHUNK = 128
GROUPS = CHUNK // L


def _body(idx_hbm, table_hbm, gamma_hbm, beta_hbm, norm_out, raw_out,
          idx_v, raw0, raw1, norm0, norm1, gamma_v, beta_v,
          sg0, sg1, sr0, sr1, sn0, sn1):
    chunks = idx_hbm.shape[1]
    nk = chunks // 2
    raws = (raw0, raw1)
    norms = (norm0, norm1)
    sgs = (sg0, sg1)
    srs = (sr0, sr1)
    sns = (sn0, sn1)

    wid = lax.axis_index("s") * NC + lax.axis_index("c")
    rows_per_worker = chunks * CHUNK
    base_w = wid * rows_per_worker

    pltpu.sync_copy(idx_hbm.at[wid], idx_v)
    pltpu.sync_copy(gamma_hbm, gamma_v)
    pltpu.sync_copy(beta_hbm, beta_v)

    iota = lax.iota(jnp.int32, L)
    row_ids = [g * L + iota for g in range(GROUPS)]
    inv_h = jnp.float32(1.0 / HIDDEN)

    def gather_chunk(i, b):
        pltpu.async_copy(table_hbm.at[idx_v.at[i]], raws[b], sgs[b])

    def process(i, b):
        q = 1 - b
        pltpu.make_async_copy(table_hbm.at[idx_v.at[i]], raws[b], sgs[b]).wait()
        base = base_w + i * CHUNK
        pltpu.async_copy(raws[b], raw_out.at[pl.ds(base, CHUNK)], srs[b])

        def col_stats(c, sc):
            sums, sumsqs = sc
            colv = jnp.full((L,), c, dtype=jnp.int32)
            ns, nq = [], []
            for g in range(GROUPS):
                x = plsc.load_gather(raws[b], [row_ids[g], colv])
                ns.append(sums[g] + x)
                nq.append(sumsqs[g] + x * x)
            return tuple(ns), tuple(nq)

        zeros = tuple(jnp.zeros((L,), jnp.float32) for _ in range(GROUPS))
        sums, sumsqs = lax.fori_loop(0, HIDDEN, col_stats, (zeros, zeros))

        means, scales = [], []
        for g in range(GROUPS):
            m = sums[g] * inv_h
            var = sumsqs[g] * inv_h - m * m
            a = var + jnp.float32(EPS)
            ib = plsc.bitcast(a, jnp.int32)
            magic = jnp.full((L,), 0x5F3759DF, dtype=jnp.int32)
            one = jnp.full((L,), 1, dtype=jnp.int32)
            y = plsc.bitcast(magic - lax.shift_right_logical(ib, one),
                             jnp.float32)
            half_a = a * jnp.float32(0.5)
            for _ in range(3):
                y = y * (jnp.float32(1.5) - half_a * y * y)
            means.append(m)
            scales.append(y)

        @pl.when(i >= 2)
        def _():
            pltpu.make_async_copy(
                norms[b], norm_out.at[pl.ds(base_w, CHUNK)], sns[b]).wait()

        def col_norm(c, cc):
            colv = jnp.full((L,), c, dtype=jnp.int32)
            gc = plsc.load_gather(gamma_v, [colv])
            bc = plsc.load_gather(beta_v, [colv])
            for g in range(GROUPS):
                x = plsc.load_gather(raws[b], [row_ids[g], colv])
                y = (x - means[g]) * scales[g] * gc + bc
                plsc.store_scatter(norms[b], [row_ids[g], colv], y)
            return cc

        lax.fori_loop(0, HIDDEN, col_norm, 0)
        pltpu.async_copy(norms[b], norm_out.at[pl.ds(base, CHUNK)], sns[b])

    gather_chunk(0, 0)

    def pair_body(k, carry):
        i0 = 2 * k

        @pl.when(k >= 1)
        def _():
            pltpu.make_async_copy(
                raws[1], raw_out.at[pl.ds(base_w, CHUNK)], srs[1]).wait()

        gather_chunk(i0 + 1, 1)
        process(i0, 0)

        pltpu.make_async_copy(
            raws[0], raw_out.at[pl.ds(base_w, CHUNK)], srs[0]).wait()

        @pl.when(k < nk - 1)
        def _():
            gather_chunk(i0 + 2, 0)

        process(i0 + 1, 1)
        return carry

    lax.fori_loop(0, nk, pair_body, 0)

    pltpu.make_async_copy(
        raws[1], raw_out.at[pl.ds(base_w, CHUNK)], srs[1]).wait()
    pltpu.make_async_copy(
        norms[0], norm_out.at[pl.ds(base_w, CHUNK)], sns[0]).wait()
    pltpu.make_async_copy(
        norms[1], norm_out.at[pl.ds(base_w, CHUNK)], sns[1]).wait()


def kernel(eye_feature, table, ln_gamma, ln_beta):
    s0, s1 = eye_feature.shape
    b = s0 * s1
    assert b % (NW * CHUNK * 2) == 0
    chunks = b // (NW * CHUNK)
    hidden = table.shape[1]

    idx = eye_feature.reshape(NW, chunks, CHUNK).astype(jnp.int32)

    run = pl.kernel(
        _body,
        out_type=(
            jax.ShapeDtypeStruct((b, hidden), jnp.float32),
            jax.ShapeDtypeStruct((b, hidden), jnp.float32),
        ),
        mesh=plsc.VectorSubcoreMesh(core_axis_name="c", subcore_axis_name="s"),
        compiler_params=pltpu.CompilerParams(needs_layout_passes=False),
        scratch_types=[
            pltpu.VMEM((chunks, CHUNK), jnp.int32),
            pltpu.VMEM((CHUNK, hidden), jnp.float32),
            pltpu.VMEM((CHUNK, hidden), jnp.float32),
            pltpu.VMEM((CHUNK, hidden), jnp.float32),
            pltpu.VMEM((CHUNK, hidden), jnp.float32),
            pltpu.VMEM((hidden,), jnp.float32),
            pltpu.VMEM((hidden,), jnp.float32),
            pltpu.SemaphoreType.DMA,
            pltpu.SemaphoreType.DMA,
            pltpu.SemaphoreType.DMA,
            pltpu.SemaphoreType.DMA,
            pltpu.SemaphoreType.DMA,
            pltpu.SemaphoreType.DMA,
        ],
    )
    norm, raw = run(idx, table, ln_gamma, ln_beta)
    return (norm.reshape(s0, s1, hidden), raw.reshape(s0, s1, hidden))


# speedup vs baseline: 3.5657x; 3.3283x over previous
import jax
import jax.numpy as jnp
from jax import lax
from jax.experimental import pallas as pl
from jax.experimental.pallas import tpu as pltpu
from jax.experimental.pallas import tpu_sc as plsc

NC = 2
NS = 16
NW = NC * NS
L = 16
HIDDEN = 128
EPS = 1e-12
CHUNK = 128
GROUPS = CHUNK // L


def _body(idx_hbm, table_hbm, gamma_hbm, beta_hbm, norm_out, raw_out,
          idx_v, raw0, raw1, norm0, norm1, gamma_v, beta_v,
          sg0, sg1, sr0, sr1, sn0, sn1):
    chunks = idx_hbm.shape[1]
    nk = chunks // 2
    raws = (raw0, raw1)
    norms = (norm0, norm1)
    sgs = (sg0, sg1)
    srs = (sr0, sr1)
    sns = (sn0, sn1)

    wid = lax.axis_index("s") * NC + lax.axis_index("c")
    rows_per_worker = chunks * CHUNK
    base_w = wid * rows_per_worker

    pltpu.sync_copy(idx_hbm.at[wid], idx_v)
    pltpu.sync_copy(gamma_hbm, gamma_v)
    pltpu.sync_copy(beta_hbm, beta_v)

    iota = lax.iota(jnp.int32, L)
    row_ids = [g * L + iota for g in range(GROUPS)]
    inv_h = jnp.float32(1.0 / HIDDEN)

    def gather_chunk(i, b):
        pltpu.async_copy(table_hbm.at[idx_v.at[i]], raws[b], sgs[b])

    def process(i, b):
        q = 1 - b
        pltpu.make_async_copy(table_hbm.at[idx_v.at[i]], raws[b], sgs[b]).wait()
        base = base_w + i * CHUNK
        pltpu.async_copy(raws[b], raw_out.at[pl.ds(base, CHUNK)], srs[b])

        def col_stats(c, sc):
            sums, sumsqs = sc
            colv = jnp.bitwise_and(c + iota, jnp.full((L,), HIDDEN - 1,
                                                      dtype=jnp.int32))
            ns, nq = [], []
            for g in range(GROUPS):
                x = plsc.load_gather(raws[b], [row_ids[g], colv])
                ns.append(sums[g] + x)
                nq.append(sumsqs[g] + x * x)
            return tuple(ns), tuple(nq)

        zeros = tuple(jnp.zeros((L,), jnp.float32) for _ in range(GROUPS))
        sums, sumsqs = lax.fori_loop(0, HIDDEN, col_stats, (zeros, zeros))

        means, scales = [], []
        for g in range(GROUPS):
            m = sums[g] * inv_h
            var = sumsqs[g] * inv_h - m * m
            a = var + jnp.float32(EPS)
            ib = plsc.bitcast(a, jnp.int32)
            magic = jnp.full((L,), 0x5F3759DF, dtype=jnp.int32)
            one = jnp.full((L,), 1, dtype=jnp.int32)
            y = plsc.bitcast(magic - lax.shift_right_logical(ib, one),
                             jnp.float32)
            half_a = a * jnp.float32(0.5)
            for _ in range(3):
                y = y * (jnp.float32(1.5) - half_a * y * y)
            means.append(m)
            scales.append(y)

        @pl.when(i >= 2)
        def _():
            pltpu.make_async_copy(
                norms[b], norm_out.at[pl.ds(base_w, CHUNK)], sns[b]).wait()

        def col_norm(c, cc):
            colv = jnp.bitwise_and(c + iota, jnp.full((L,), HIDDEN - 1,
                                                      dtype=jnp.int32))
            gc = plsc.load_gather(gamma_v, [colv])
            bc = plsc.load_gather(beta_v, [colv])
            for g in range(GROUPS):
                x = plsc.load_gather(raws[b], [row_ids[g], colv])
                y = (x - means[g]) * scales[g] * gc + bc
                plsc.store_scatter(norms[b], [row_ids[g], colv], y)
            return cc

        lax.fori_loop(0, HIDDEN, col_norm, 0)
        pltpu.async_copy(norms[b], norm_out.at[pl.ds(base, CHUNK)], sns[b])

    gather_chunk(0, 0)

    def pair_body(k, carry):
        i0 = 2 * k

        @pl.when(k >= 1)
        def _():
            pltpu.make_async_copy(
                raws[1], raw_out.at[pl.ds(base_w, CHUNK)], srs[1]).wait()

        gather_chunk(i0 + 1, 1)
        process(i0, 0)

        pltpu.make_async_copy(
            raws[0], raw_out.at[pl.ds(base_w, CHUNK)], srs[0]).wait()

        @pl.when(k < nk - 1)
        def _():
            gather_chunk(i0 + 2, 0)

        process(i0 + 1, 1)
        return carry

    lax.fori_loop(0, nk, pair_body, 0)

    pltpu.make_async_copy(
        raws[1], raw_out.at[pl.ds(base_w, CHUNK)], srs[1]).wait()
    pltpu.make_async_copy(
        norms[0], norm_out.at[pl.ds(base_w, CHUNK)], sns[0]).wait()
    pltpu.make_async_copy(
        norms[1], norm_out.at[pl.ds(base_w, CHUNK)], sns[1]).wait()


def kernel(eye_feature, table, ln_gamma, ln_beta):
    s0, s1 = eye_feature.shape
    b = s0 * s1
    assert b % (NW * CHUNK * 2) == 0
    chunks = b // (NW * CHUNK)
    hidden = table.shape[1]

    idx = eye_feature.reshape(NW, chunks, CHUNK).astype(jnp.int32)

    run = pl.kernel(
        _body,
        out_type=(
            jax.ShapeDtypeStruct((b, hidden), jnp.float32),
            jax.ShapeDtypeStruct((b, hidden), jnp.float32),
        ),
        mesh=plsc.VectorSubcoreMesh(core_axis_name="c", subcore_axis_name="s"),
        compiler_params=pltpu.CompilerParams(needs_layout_passes=False),
        scratch_types=[
            pltpu.VMEM((chunks, CHUNK), jnp.int32),
            pltpu.VMEM((CHUNK, hidden), jnp.float32),
            pltpu.VMEM((CHUNK, hidden), jnp.float32),
            pltpu.VMEM((CHUNK, hidden), jnp.float32),
            pltpu.VMEM((CHUNK, hidden), jnp.float32),
            pltpu.VMEM((hidden,), jnp.float32),
            pltpu.VMEM((hidden,), jnp.float32),
            pltpu.SemaphoreType.DMA,
            pltpu.SemaphoreType.DMA,
            pltpu.SemaphoreType.DMA,
            pltpu.SemaphoreType.DMA,
            pltpu.SemaphoreType.DMA,
            pltpu.SemaphoreType.DMA,
        ],
    )
    norm, raw = run(idx, table, ln_gamma, ln_beta)
    return (norm.reshape(s0, s1, hidden), raw.reshape(s0, s1, hidden))


# speedup vs baseline: 8.7055x; 2.4415x over previous
import jax
import jax.numpy as jnp
from jax import lax
from jax.experimental import pallas as pl
from jax.experimental.pallas import tpu as pltpu
from jax.experimental.pallas import tpu_sc as plsc

NC = 2
NS = 16
NW = NC * NS
HIDDEN = 128
EPS = 1e-12
CHUNK = 128
LN_BLK = 2048


def _gather_body(idx_hbm, table_hbm, raw_out,
                 idx_v, buf0, buf1, sg0, sg1, sw0, sw1):
    chunks = idx_hbm.shape[1]
    nk = chunks // 2
    bufs = (buf0, buf1)
    sgs = (sg0, sg1)
    sws = (sw0, sw1)

    wid = lax.axis_index("s") * NC + lax.axis_index("c")
    base_w = wid * chunks * CHUNK

    pltpu.sync_copy(idx_hbm.at[wid], idx_v)

    def gather_chunk(i, b):
        pltpu.async_copy(table_hbm.at[idx_v.at[i]], bufs[b], sgs[b])

    def write_chunk(i, b):
        pltpu.make_async_copy(table_hbm.at[idx_v.at[i]], bufs[b], sgs[b]).wait()
        pltpu.async_copy(bufs[b], raw_out.at[pl.ds(base_w + i * CHUNK, CHUNK)],
                         sws[b])

    def wait_write(b):
        pltpu.make_async_copy(
            bufs[b], raw_out.at[pl.ds(base_w, CHUNK)], sws[b]).wait()

    gather_chunk(0, 0)

    def pair_body(k, carry):
        i0 = 2 * k

        @pl.when(k >= 1)
        def _():
            wait_write(1)

        gather_chunk(i0 + 1, 1)
        write_chunk(i0, 0)

        wait_write(0)

        @pl.when(k < nk - 1)
        def _():
            gather_chunk(i0 + 2, 0)

        write_chunk(i0 + 1, 1)
        return carry

    lax.fori_loop(0, nk, pair_body, 0)
    wait_write(1)


def _ln_body(raw_ref, gamma_ref, beta_ref, norm_ref):
    x = raw_ref[...]
    mean = jnp.mean(x, axis=-1, keepdims=True)
    xc = x - mean
    var = jnp.mean(xc * xc, axis=-1, keepdims=True)
    inv = lax.rsqrt(var + jnp.float32(EPS))
    norm_ref[...] = xc * inv * gamma_ref[...] + beta_ref[...]


def kernel(eye_feature, table, ln_gamma, ln_beta):
    s0, s1 = eye_feature.shape
    b = s0 * s1
    assert b % (NW * CHUNK * 2) == 0
    chunks = b // (NW * CHUNK)
    hidden = table.shape[1]

    idx = eye_feature.reshape(NW, chunks, CHUNK).astype(jnp.int32)

    gather = pl.kernel(
        _gather_body,
        out_type=jax.ShapeDtypeStruct((b, hidden), jnp.float32),
        mesh=plsc.VectorSubcoreMesh(core_axis_name="c", subcore_axis_name="s"),
        compiler_params=pltpu.CompilerParams(needs_layout_passes=False),
        scratch_types=[
            pltpu.VMEM((chunks, CHUNK), jnp.int32),
            pltpu.VMEM((CHUNK, hidden), jnp.float32),
            pltpu.VMEM((CHUNK, hidden), jnp.float32),
            pltpu.SemaphoreType.DMA,
            pltpu.SemaphoreType.DMA,
            pltpu.SemaphoreType.DMA,
            pltpu.SemaphoreType.DMA,
        ],
    )
    raw = gather(idx, table)

    norm = pl.pallas_call(
        _ln_body,
        grid=(b // LN_BLK,),
        in_specs=[
            pl.BlockSpec((LN_BLK, hidden), lambda i: (i, 0)),
            pl.BlockSpec((hidden,), lambda i: (0,)),
            pl.BlockSpec((hidden,), lambda i: (0,)),
        ],
        out_specs=pl.BlockSpec((LN_BLK, hidden), lambda i: (i, 0)),
        out_shape=jax.ShapeDtypeStruct((b, hidden), jnp.float32),
    )(raw, ln_gamma, ln_beta)

    return (norm.reshape(s0, s1, hidden), raw.reshape(s0, s1, hidden))


# speedup vs baseline: 11.1195x; 1.2773x over previous
import jax
import jax.numpy as jnp
from jax import lax
from jax.experimental import pallas as pl
from jax.experimental.pallas import tpu as pltpu
from jax.experimental.pallas import tpu_sc as plsc

NC = 2
NS = 16
NW = NC * NS
HIDDEN = 128
EPS = 1e-12
CHUNK = 128
LN_BLK = 8192


def _gather_body(idx_hbm, table_hbm, raw_out,
                 idx_v, buf0, buf1, sg0, sg1, sw0, sw1):
    chunks = idx_hbm.shape[1]
    nk = chunks // 2
    bufs = (buf0, buf1)
    sgs = (sg0, sg1)
    sws = (sw0, sw1)

    wid = lax.axis_index("s") * NC + lax.axis_index("c")
    base_w = wid * chunks * CHUNK

    pltpu.sync_copy(idx_hbm.at[wid], idx_v)

    def gather_chunk(i, b):
        pltpu.async_copy(table_hbm.at[idx_v.at[i]], bufs[b], sgs[b])

    def write_chunk(i, b):
        pltpu.make_async_copy(table_hbm.at[idx_v.at[i]], bufs[b], sgs[b]).wait()
        pltpu.async_copy(bufs[b], raw_out.at[pl.ds(base_w + i * CHUNK, CHUNK)],
                         sws[b])

    def wait_write(b):
        pltpu.make_async_copy(
            bufs[b], raw_out.at[pl.ds(base_w, CHUNK)], sws[b]).wait()

    gather_chunk(0, 0)

    def pair_body(k, carry):
        i0 = 2 * k

        @pl.when(k >= 1)
        def _():
            wait_write(1)

        gather_chunk(i0 + 1, 1)
        write_chunk(i0, 0)

        wait_write(0)

        @pl.when(k < nk - 1)
        def _():
            gather_chunk(i0 + 2, 0)

        write_chunk(i0 + 1, 1)
        return carry

    lax.fori_loop(0, nk, pair_body, 0)
    wait_write(1)


def _ln_body(raw_ref, gamma_ref, beta_ref, norm_ref):
    x = raw_ref[...]
    mean = jnp.mean(x, axis=-1, keepdims=True)
    xc = x - mean
    var = jnp.mean(xc * xc, axis=-1, keepdims=True)
    inv = lax.rsqrt(var + jnp.float32(EPS))
    norm_ref[...] = xc * inv * gamma_ref[...] + beta_ref[...]


def kernel(eye_feature, table, ln_gamma, ln_beta):
    s0, s1 = eye_feature.shape
    b = s0 * s1
    assert b % (NW * CHUNK * 2) == 0
    chunks = b // (NW * CHUNK)
    hidden = table.shape[1]

    idx = eye_feature.reshape(NW, chunks, CHUNK).astype(jnp.int32)

    gather = pl.kernel(
        _gather_body,
        out_type=jax.ShapeDtypeStruct((b, hidden), jnp.float32),
        mesh=plsc.VectorSubcoreMesh(core_axis_name="c", subcore_axis_name="s"),
        compiler_params=pltpu.CompilerParams(needs_layout_passes=False),
        scratch_types=[
            pltpu.VMEM((chunks, CHUNK), jnp.int32),
            pltpu.VMEM((CHUNK, hidden), jnp.float32),
            pltpu.VMEM((CHUNK, hidden), jnp.float32),
            pltpu.SemaphoreType.DMA,
            pltpu.SemaphoreType.DMA,
            pltpu.SemaphoreType.DMA,
            pltpu.SemaphoreType.DMA,
        ],
    )
    raw = gather(idx, table)

    norm = pl.pallas_call(
        _ln_body,
        grid=(b // LN_BLK,),
        in_specs=[
            pl.BlockSpec((LN_BLK, hidden), lambda i: (i, 0)),
            pl.BlockSpec((hidden,), lambda i: (0,)),
            pl.BlockSpec((hidden,), lambda i: (0,)),
        ],
        out_specs=pl.BlockSpec((LN_BLK, hidden), lambda i: (i, 0)),
        out_shape=jax.ShapeDtypeStruct((b, hidden), jnp.float32),
    )(raw, ln_gamma, ln_beta)

    return (norm.reshape(s0, s1, hidden), raw.reshape(s0, s1, hidden))


# speedup vs baseline: 11.5593x; 1.0396x over previous
import jax
import jax.numpy as jnp
from jax import lax
from jax.experimental import pallas as pl
from jax.experimental.pallas import tpu as pltpu
from jax.experimental.pallas import tpu_sc as plsc

NC = 2
NS = 16
NW = NC * NS
HIDDEN = 128
EPS = 1e-12
CHUNK = 128
LN_BLK = 16384


def _gather_body(idx_hbm, table_hbm, raw_out,
                 idx_v, buf0, buf1, sg0, sg1, sw0, sw1):
    chunks = idx_hbm.shape[1]
    nk = chunks // 2
    bufs = (buf0, buf1)
    sgs = (sg0, sg1)
    sws = (sw0, sw1)

    wid = lax.axis_index("s") * NC + lax.axis_index("c")
    base_w = wid * chunks * CHUNK

    pltpu.sync_copy(idx_hbm.at[wid], idx_v)

    def gather_chunk(i, b):
        pltpu.async_copy(table_hbm.at[idx_v.at[i]], bufs[b], sgs[b])

    def write_chunk(i, b):
        pltpu.make_async_copy(table_hbm.at[idx_v.at[i]], bufs[b], sgs[b]).wait()
        pltpu.async_copy(bufs[b], raw_out.at[pl.ds(base_w + i * CHUNK, CHUNK)],
                         sws[b])

    def wait_write(b):
        pltpu.make_async_copy(
            bufs[b], raw_out.at[pl.ds(base_w, CHUNK)], sws[b]).wait()

    gather_chunk(0, 0)

    def pair_body(k, carry):
        i0 = 2 * k

        @pl.when(k >= 1)
        def _():
            wait_write(1)

        gather_chunk(i0 + 1, 1)
        write_chunk(i0, 0)

        wait_write(0)

        @pl.when(k < nk - 1)
        def _():
            gather_chunk(i0 + 2, 0)

        write_chunk(i0 + 1, 1)
        return carry

    lax.fori_loop(0, nk, pair_body, 0)
    wait_write(1)


def _ln_body(raw_ref, gamma_ref, beta_ref, norm_ref):
    x = raw_ref[...]
    mean = jnp.mean(x, axis=-1, keepdims=True)
    xc = x - mean
    var = jnp.mean(xc * xc, axis=-1, keepdims=True)
    inv = lax.rsqrt(var + jnp.float32(EPS))
    norm_ref[...] = xc * inv * gamma_ref[...] + beta_ref[...]


def kernel(eye_feature, table, ln_gamma, ln_beta):
    s0, s1 = eye_feature.shape
    b = s0 * s1
    assert b % (NW * CHUNK * 2) == 0
    chunks = b // (NW * CHUNK)
    hidden = table.shape[1]

    idx = eye_feature.reshape(NW, chunks, CHUNK).astype(jnp.int32)

    gather = pl.kernel(
        _gather_body,
        out_type=jax.ShapeDtypeStruct((b, hidden), jnp.float32),
        mesh=plsc.VectorSubcoreMesh(core_axis_name="c", subcore_axis_name="s"),
        compiler_params=pltpu.CompilerParams(needs_layout_passes=False),
        scratch_types=[
            pltpu.VMEM((chunks, CHUNK), jnp.int32),
            pltpu.VMEM((CHUNK, hidden), jnp.float32),
            pltpu.VMEM((CHUNK, hidden), jnp.float32),
            pltpu.SemaphoreType.DMA,
            pltpu.SemaphoreType.DMA,
            pltpu.SemaphoreType.DMA,
            pltpu.SemaphoreType.DMA,
        ],
    )
    raw = gather(idx, table)

    norm = pl.pallas_call(
        _ln_body,
        grid=(b // LN_BLK,),
        in_specs=[
            pl.BlockSpec((LN_BLK, hidden), lambda i: (i, 0)),
            pl.BlockSpec((hidden,), lambda i: (0,)),
            pl.BlockSpec((hidden,), lambda i: (0,)),
        ],
        out_specs=pl.BlockSpec((LN_BLK, hidden), lambda i: (i, 0)),
        out_shape=jax.ShapeDtypeStruct((b, hidden), jnp.float32),
    )(raw, ln_gamma, ln_beta)

    return (norm.reshape(s0, s1, hidden), raw.reshape(s0, s1, hidden))


# speedup vs baseline: 11.7434x; 1.0159x over previous
import jax
import jax.numpy as jnp
from jax import lax
from jax.experimental import pallas as pl
from jax.experimental.pallas import tpu as pltpu
from jax.experimental.pallas import tpu_sc as plsc

NC = 2
NS = 16
NW = NC * NS
HIDDEN = 128
EPS = 1e-12
CHUNK = 128
LN_BLK = 20480


def _gather_body(idx_hbm, table_hbm, raw_out,
                 idx_v, buf0, buf1, sg0, sg1, sw0, sw1):
    chunks = idx_hbm.shape[1]
    nk = chunks // 2
    bufs = (buf0, buf1)
    sgs = (sg0, sg1)
    sws = (sw0, sw1)

    wid = lax.axis_index("s") * NC + lax.axis_index("c")
    base_w = wid * chunks * CHUNK

    pltpu.sync_copy(idx_hbm.at[wid], idx_v)

    def gather_chunk(i, b):
        pltpu.async_copy(table_hbm.at[idx_v.at[i]], bufs[b], sgs[b])

    def write_chunk(i, b):
        pltpu.make_async_copy(table_hbm.at[idx_v.at[i]], bufs[b], sgs[b]).wait()
        pltpu.async_copy(bufs[b], raw_out.at[pl.ds(base_w + i * CHUNK, CHUNK)],
                         sws[b])

    def wait_write(b):
        pltpu.make_async_copy(
            bufs[b], raw_out.at[pl.ds(base_w, CHUNK)], sws[b]).wait()

    gather_chunk(0, 0)

    def pair_body(k, carry):
        i0 = 2 * k

        @pl.when(k >= 1)
        def _():
            wait_write(1)

        gather_chunk(i0 + 1, 1)
        write_chunk(i0, 0)

        wait_write(0)

        @pl.when(k < nk - 1)
        def _():
            gather_chunk(i0 + 2, 0)

        write_chunk(i0 + 1, 1)
        return carry

    lax.fori_loop(0, nk, pair_body, 0)
    wait_write(1)


def _ln_body(raw_ref, gamma_ref, beta_ref, norm_ref):
    x = raw_ref[...]
    mean = jnp.mean(x, axis=-1, keepdims=True)
    xc = x - mean
    var = jnp.mean(xc * xc, axis=-1, keepdims=True)
    inv = lax.rsqrt(var + jnp.float32(EPS))
    norm_ref[...] = xc * inv * gamma_ref[...] + beta_ref[...]


def kernel(eye_feature, table, ln_gamma, ln_beta):
    s0, s1 = eye_feature.shape
    b = s0 * s1
    assert b % (NW * CHUNK * 2) == 0
    chunks = b // (NW * CHUNK)
    hidden = table.shape[1]

    idx = eye_feature.reshape(NW, chunks, CHUNK).astype(jnp.int32)

    gather = pl.kernel(
        _gather_body,
        out_type=jax.ShapeDtypeStruct((b, hidden), jnp.float32),
        mesh=plsc.VectorSubcoreMesh(core_axis_name="c", subcore_axis_name="s"),
        compiler_params=pltpu.CompilerParams(needs_layout_passes=False),
        scratch_types=[
            pltpu.VMEM((chunks, CHUNK), jnp.int32),
            pltpu.VMEM((CHUNK, hidden), jnp.float32),
            pltpu.VMEM((CHUNK, hidden), jnp.float32),
            pltpu.SemaphoreType.DMA,
            pltpu.SemaphoreType.DMA,
            pltpu.SemaphoreType.DMA,
            pltpu.SemaphoreType.DMA,
        ],
    )
    raw = gather(idx, table)

    norm = pl.pallas_call(
        _ln_body,
        grid=(b // LN_BLK,),
        in_specs=[
            pl.BlockSpec((LN_BLK, hidden), lambda i: (i, 0)),
            pl.BlockSpec((hidden,), lambda i: (0,)),
            pl.BlockSpec((hidden,), lambda i: (0,)),
        ],
        out_specs=pl.BlockSpec((LN_BLK, hidden), lambda i: (i, 0)),
        out_shape=jax.ShapeDtypeStruct((b, hidden), jnp.float32),
    )(raw, ln_gamma, ln_beta)

    return (norm.reshape(s0, s1, hidden), raw.reshape(s0, s1, hidden))
